# baseline (speedup 1.0000x reference)
import jax
import jax.numpy as jnp
from jax import lax
from jax.experimental import pallas as pl
from jax.experimental.pallas import tpu as pltpu
from jax.experimental.pallas import tpu_sc as plsc


def _build_sc_kernel(B, N, SPECIAL, K, D, NUM_EDGE, NUM_SPATIAL):
    NS = N + SPECIAL
    NW = 32
    tasks_per_w = (B * N * 2) // NW
    brows_per_w = (B * SPECIAL) // NW
    LQ = D // 16

    CFG = (
        dict(jj_n=68, spd_off=0, spd_n=68, spd_lead=0,
             e_chunks=((0, 0, 128, 0), (1, 0, 76, 128)), e_n=204, e_lead=0,
             acc_n=72, acc_lead=SPECIAL, out_col=0),
        dict(jj_n=60, spd_off=64, spd_n=64, spd_lead=4,
             e_chunks=((1, 72, 56, 0), (2, 0, 128, 56)), e_n=184, e_lead=4,
             acc_n=60, acc_lead=0, out_col=72),
    )

    mesh = plsc.VectorSubcoreMesh(core_axis_name="c", subcore_axis_name="s")

    @jax.jit
    def run(sp4, ef4, we_pad, ws_pad, W_vnode):
        @pl.kernel(
            out_type=jax.ShapeDtypeStruct((B, NS, NS, D), jnp.float32),
            mesh=mesh,
            scratch_types=[
                pltpu.VMEM((2, 8, N), jnp.int32),
                pltpu.VMEM((2, K, N), jnp.int32),
                pltpu.VMEM((CFG[0]["spd_n"], 128), jnp.float32),
                pltpu.VMEM((CFG[1]["spd_n"], 128), jnp.float32),
                pltpu.VMEM((CFG[0]["e_n"], 128), jnp.float32),
                pltpu.VMEM((CFG[1]["e_n"], 128), jnp.float32),
                pltpu.VMEM((CFG[0]["acc_n"], D), jnp.float32),
                pltpu.VMEM((CFG[1]["acc_n"], D), jnp.float32),
                pltpu.VMEM((NS, D), jnp.float32),
                pltpu.VMEM((1, D), jnp.float32),
                pltpu.VMEM_SHARED((NUM_SPATIAL, 128), jnp.float32),
                pltpu.VMEM_SHARED((NUM_EDGE, 128), jnp.float32),
                pltpu.SemaphoreType.DMA,
                pltpu.SemaphoreType.DMA,
                pltpu.SemaphoreType.DMA,
                pltpu.SemaphoreType.DMA,
                pltpu.SemaphoreType.DMA,
                pltpu.SemaphoreType.DMA,
                pltpu.SemaphoreType.DMA,
            ],
        )
        def k(sp_hbm, ef_hbm, we_hbm, ws_hbm, wv_hbm, out_hbm,
              spv2, efv2, bufs_a, bufs_b, bufe_a, bufe_b, acc_a, acc_b,
              vrow, vnv, ws_sh, we_sh,
              sem_i0, sem_i1, sem_g0, sem_g1, sem_o0, sem_o1, sem_b):
            sid = lax.axis_index("s")
            wid = sid * 2 + lax.axis_index("c")
            sem_i = (sem_i0, sem_i1)
            sem_g = (sem_g0, sem_g1)
            sem_o = (sem_o0, sem_o1)
            bufs = (bufs_a, bufs_b)
            bufe = (bufe_a, bufe_b)
            acc = (acc_a, acc_b)

            def task_bi(t):
                r = (wid * tasks_per_w + t) // 2
                b = r // N
                return b, r - b * N

            def issue_idx(t, p):
                b, i = task_bi(t)
                pltpu.async_copy(sp_hbm.at[b, i // 8], spv2.at[p], sem_i[p])
                pltpu.async_copy(ef_hbm.at[b, i], efv2.at[p], sem_i[p])

            def wait_idx(p):
                pltpu.make_async_copy(sp_hbm.at[0, 0], spv2.at[p], sem_i[p]).wait()
                pltpu.make_async_copy(ef_hbm.at[0, 0], efv2.at[p], sem_i[p]).wait()

            def gather_copies(t, p, mk):
                c = CFG[p]
                _, i = task_bi(t)
                ops = [mk(ws_sh.at[spv2.at[p].at[i % 8, pl.ds(c["spd_off"], c["spd_n"])]],
                          bufs[p], sem_g[p])]
                for (lrow, loff, n, doff) in c["e_chunks"]:
                    ops.append(mk(we_sh.at[efv2.at[p].at[lrow, pl.ds(loff, n)]],
                                  bufe[p].at[pl.ds(doff, n)], sem_g[p]))
                return ops

            def issue_gathers(t, p):
                gather_copies(t, p, pltpu.async_copy)

            def wait_gathers(p):
                for op in gather_copies(0, p, pltpu.make_async_copy):
                    op.wait()

            def out_copy(t, p, mk):
                c = CFG[p]
                b, i = task_bi(t)
                return mk(acc[p], out_hbm.at[b, i + SPECIAL,
                                             pl.ds(c["out_col"], c["acc_n"])],
                          sem_o[p])

            def combine_out(t, p):
                c = CFG[p]
                be = bufe[p]
                bs = bufs[p]
                ac = acc[p]
                e_lead, s_lead, a_lead = c["e_lead"], c["spd_lead"], c["acc_lead"]

                @plsc.parallel_loop(0, c["jj_n"], unroll=4)
                def _(jj):
                    t0 = jj * K + e_lead
                    for q in range(LQ):
                        s = pl.ds(q * 16, 16)
                        e = be[t0, s] + be[t0 + 1, s] + be[t0 + 2, s]
                        ac[jj + a_lead, s] = bs[jj + s_lead, s] + e * (1.0 / 3.0)

                out_copy(t, p, pltpu.async_copy)

            def wait_out(p):
                out_copy(0, p, pltpu.make_async_copy).wait()

            pltpu.sync_copy(ws_hbm.at[pl.ds(sid * (NUM_SPATIAL // 16), NUM_SPATIAL // 16)],
                            ws_sh.at[pl.ds(sid * (NUM_SPATIAL // 16), NUM_SPATIAL // 16)])
            pltpu.sync_copy(we_hbm.at[pl.ds(sid * (NUM_EDGE // 16), NUM_EDGE // 16)],
                            we_sh.at[pl.ds(sid * (NUM_EDGE // 16), NUM_EDGE // 16)])
            plsc.subcore_barrier()

            pltpu.async_copy(wv_hbm, vnv, sem_i0).wait()

            @plsc.parallel_loop(0, NS, unroll=4)
            def _(r):
                for q in range(LQ):
                    s = pl.ds(q * 16, 16)
                    vrow[r, s] = vnv[0, s]

            @plsc.parallel_loop(0, SPECIAL)
            def _(r):
                for q in range(LQ):
                    s = pl.ds(q * 16, 16)
                    acc_a[r, s] = vnv[0, s]

            for p in range(brows_per_w):
                r2 = wid * brows_per_w + p
                b2 = r2 // SPECIAL
                i2 = r2 - b2 * SPECIAL
                pltpu.async_copy(vrow, out_hbm.at[b2, i2], sem_b)

            issue_idx(0, 0)
            issue_idx(1, 1)
            wait_idx(0)
            issue_gathers(0, 0)

            @pl.loop(0, tasks_per_w, step=2)
            def _(g0):
                g2 = (g0 + 2) % tasks_per_w
                g3 = (g0 + 3) % tasks_per_w
                wait_idx(1)
                issue_gathers(g0 + 1, 1)
                wait_gathers(0)
                issue_idx(g2, 0)

                @pl.when(g0 >= 2)
                def _():
                    wait_out(0)

                combine_out(g0, 0)
                wait_gathers(1)
                wait_idx(0)
                issue_gathers(g2, 0)
                issue_idx(g3, 1)

                @pl.when(g0 >= 2)
                def _():
                    wait_out(1)

                combine_out(g0 + 1, 1)

            wait_gathers(0)
            wait_idx(1)
            wait_out(0)
            wait_out(1)
            for p in range(brows_per_w):
                pltpu.make_async_copy(vrow, out_hbm.at[0, 0], sem_b).wait()

        return k(sp4, ef4, we_pad, ws_pad, W_vnode)

    return run


def kernel(shortest_path, edge_feat, seg_feat, graph_attn_bias, W_edge, W_spd, W_vnode):
    B, N, _ = shortest_path.shape
    K = edge_feat.shape[-1]
    SPECIAL = 2 + seg_feat.shape[1]
    D = W_vnode.shape[-1]
    ws_pad = jnp.pad(W_spd, ((0, 0), (0, 128 - D)))
    we_pad = jnp.pad(W_edge, ((0, 0), (0, 128 - D)))
    sp4 = shortest_path.reshape(B, N // 8, 8, N)
    ef4 = edge_feat.reshape(B, N, K, N)
    run = _build_sc_kernel(B, N, SPECIAL, K, D, W_edge.shape[0], W_spd.shape[0])
    return run(sp4, ef4, we_pad, ws_pad, W_vnode)


# speedup vs baseline: 1.1223x; 1.1223x over previous
import jax
import jax.numpy as jnp
from jax import lax
from jax.experimental import pallas as pl
from jax.experimental.pallas import tpu as pltpu
from jax.experimental.pallas import tpu_sc as plsc


def _build_sc_kernel(B, N, SPECIAL, K, D, NUM_EDGE, NUM_SPATIAL):
    NS = N + SPECIAL
    NW = 32
    rows_per_w = (B * N) // NW
    brows_per_w = (B * SPECIAL) // NW
    LQ = D // 16

    mesh = plsc.VectorSubcoreMesh(core_axis_name="c", subcore_axis_name="s")

    @jax.jit
    def run(sp, ef4, W_edge, W_spd, W_vnode):
        @pl.kernel(
            out_type=jax.ShapeDtypeStruct((B, NS, NS, D), jnp.float32),
            mesh=mesh,
            compiler_params=pltpu.CompilerParams(use_tc_tiling_on_sc=False),
            scratch_types=[
                pltpu.VMEM((2, N), jnp.int32),
                pltpu.VMEM((2, K, N), jnp.int32),
                pltpu.VMEM((2, N, D), jnp.float32),
                pltpu.VMEM((2, K * N, D), jnp.float32),
                pltpu.VMEM((2, NS, D), jnp.float32),
                pltpu.VMEM((NS, D), jnp.float32),
                pltpu.VMEM((1, D), jnp.float32),
                pltpu.VMEM_SHARED((NUM_SPATIAL, D), jnp.float32),
                pltpu.VMEM_SHARED((NUM_EDGE, D), jnp.float32),
                pltpu.SemaphoreType.DMA,
                pltpu.SemaphoreType.DMA,
                pltpu.SemaphoreType.DMA,
                pltpu.SemaphoreType.DMA,
                pltpu.SemaphoreType.DMA,
                pltpu.SemaphoreType.DMA,
            ],
        )
        def k(sp_hbm, ef_hbm, we_hbm, ws_hbm, wv_hbm, out_hbm,
              spv2, efv2, bufs2, bufe2, acc2, vrow, vnv, ws_sh, we_sh,
              sem_i0, sem_i1, sem_g0, sem_g1, sem_o0, sem_o1):
            sid = lax.axis_index("s")
            wid = sid * 2 + lax.axis_index("c")
            sem_i = (sem_i0, sem_i1)
            sem_g = (sem_g0, sem_g1)
            sem_o = (sem_o0, sem_o1)

            def row_bi(g):
                r = wid * rows_per_w + g
                b = r // N
                return b, r - b * N

            def issue_idx(g, p):
                b, i = row_bi(g)
                pltpu.async_copy(sp_hbm.at[b, i], spv2.at[p], sem_i[p])
                pltpu.async_copy(ef_hbm.at[b, i], efv2.at[p], sem_i[p])

            def wait_idx(p):
                pltpu.make_async_copy(sp_hbm.at[0, 0], spv2.at[p], sem_i[p]).wait()
                pltpu.make_async_copy(ef_hbm.at[0, 0], efv2.at[p], sem_i[p]).wait()

            def issue_gathers(p):
                pltpu.async_copy(ws_sh.at[spv2.at[p]], bufs2.at[p], sem_g[p])
                for c in range(K):
                    pltpu.async_copy(we_sh.at[efv2.at[p].at[c]],
                                     bufe2.at[p].at[pl.ds(c * N, N)], sem_g[p])

            def wait_gathers(p):
                pltpu.make_async_copy(ws_sh.at[spv2.at[p]], bufs2.at[p],
                                      sem_g[p]).wait()
                for c in range(K):
                    pltpu.make_async_copy(we_sh.at[efv2.at[p].at[c]],
                                          bufe2.at[p].at[pl.ds(c * N, N)],
                                          sem_g[p]).wait()

            def combine_out(g, p):
                bufe = bufe2.at[p]
                bufs = bufs2.at[p]
                acc = acc2.at[p]

                @plsc.parallel_loop(0, N, unroll=4)
                def _(j):
                    t0 = j * K
                    a = j + SPECIAL
                    for q in range(LQ):
                        s = pl.ds(q * 16, 16)
                        e = bufe[t0, s] + bufe[t0 + 1, s] + bufe[t0 + 2, s]
                        acc[a, s] = bufs[j, s] + e * (1.0 / 3.0)

                b, i = row_bi(g)
                pltpu.async_copy(acc2.at[p], out_hbm.at[b, i + SPECIAL], sem_o[p])

            def wait_out(p):
                pltpu.make_async_copy(acc2.at[p], out_hbm.at[0, 0], sem_o[p]).wait()

            pltpu.sync_copy(ws_hbm.at[pl.ds(sid * (NUM_SPATIAL // 16), NUM_SPATIAL // 16)],
                            ws_sh.at[pl.ds(sid * (NUM_SPATIAL // 16), NUM_SPATIAL // 16)])
            pltpu.sync_copy(we_hbm.at[pl.ds(sid * (NUM_EDGE // 16), NUM_EDGE // 16)],
                            we_sh.at[pl.ds(sid * (NUM_EDGE // 16), NUM_EDGE // 16)])
            plsc.subcore_barrier()

            pltpu.async_copy(wv_hbm, vnv, sem_i0).wait()

            @plsc.parallel_loop(0, NS, unroll=4)
            def _(r):
                for q in range(LQ):
                    s = pl.ds(q * 16, 16)
                    vrow[r, s] = vnv[0, s]

            @plsc.parallel_loop(0, SPECIAL)
            def _(r):
                for q in range(LQ):
                    s = pl.ds(q * 16, 16)
                    acc2[0, r, s] = vnv[0, s]
                    acc2[1, r, s] = vnv[0, s]

            for p in range(brows_per_w):
                r2 = wid * brows_per_w + p
                b2 = r2 // SPECIAL
                i2 = r2 - b2 * SPECIAL
                pltpu.async_copy(vrow, out_hbm.at[b2, i2], sem_o[p])

            issue_idx(0, 0)
            issue_idx(1, 1)
            wait_idx(0)
            issue_gathers(0)

            @pl.loop(0, rows_per_w, step=2)
            def _(g0):
                g2 = (g0 + 2) % rows_per_w
                g3 = (g0 + 3) % rows_per_w
                wait_idx(1)
                issue_gathers(1)
                wait_gathers(0)
                issue_idx(g2, 0)
                wait_out(0)
                combine_out(g0, 0)
                wait_gathers(1)
                wait_idx(0)
                issue_gathers(0)
                issue_idx(g3, 1)
                wait_out(1)
                combine_out(g0 + 1, 1)

            wait_gathers(0)
            wait_idx(1)
            wait_out(0)
            wait_out(1)

        return k(sp, ef4, W_edge, W_spd, W_vnode)

    return run


def kernel(shortest_path, edge_feat, seg_feat, graph_attn_bias, W_edge, W_spd, W_vnode):
    B, N, _ = shortest_path.shape
    K = edge_feat.shape[-1]
    SPECIAL = 2 + seg_feat.shape[1]
    D = W_vnode.shape[-1]
    ef4 = edge_feat.reshape(B, N, K, N)
    run = _build_sc_kernel(B, N, SPECIAL, K, D, W_edge.shape[0], W_spd.shape[0])
    return run(shortest_path, ef4, W_edge, W_spd, W_vnode)
